# baseline (speedup 1.0000x reference)
import jax
import jax.numpy as jnp
from jax import lax
from jax.experimental import pallas as pl

D = 768
DH = 3072
DK = 64
DV = 64
NH = 12
SEGF = 2048
KSEL = 256
SEGA = 256
SHID = 256
CH = 256


def _route_kernel(x_ref, ws1_ref, bs1_ref, ws2_ref, bs2_ref,
                  xsel_ref, mask_ref, slot_ref):
    x2 = x_ref[0]
    h = jnp.maximum(
        jnp.dot(x2, ws1_ref[...], preferred_element_type=jnp.float32)
        + bs1_ref[...], 0.0)
    scol = (jnp.dot(h, ws2_ref[...], preferred_element_type=jnp.float32)
            + bs2_ref[...])

    nch = SEGF // CH
    eye = (lax.broadcasted_iota(jnp.int32, (CH, CH), 0) ==
           lax.broadcasted_iota(jnp.int32, (CH, CH), 1)).astype(jnp.float32)

    def _row(vcol):
        parts = [
            lax.dot_general(vcol[c * CH:(c + 1) * CH], eye,
                            (((0,), (0,)), ((), ())),
                            preferred_element_type=jnp.float32)
            for c in range(nch)
        ]
        return jnp.concatenate(parts, axis=1)

    eye_b = eye.astype(jnp.bfloat16)

    def _rowb(vcol_b):
        parts = [
            lax.dot_general(vcol_b[c * CH:(c + 1) * CH], eye_b,
                            (((0,), (0,)), ((), ())),
                            preferred_element_type=jnp.float32)
            for c in range(nch)
        ]
        return jnp.concatenate(parts, axis=1)

    sa = scol.astype(jnp.bfloat16)
    sbp = scol - sa.astype(jnp.float32)
    sb = sbp.astype(jnp.bfloat16)
    sc2 = (sbp - sb.astype(jnp.float32)).astype(jnp.bfloat16)
    srow = (_rowb(sb) + _rowb(sc2)) + _rowb(sa)

    irow = lax.broadcasted_iota(jnp.int32, (1, SEGF), 1)

    mask_chunks = []
    for c in range(nch):
        sc = scol[c * CH:(c + 1) * CH]
        ic = lax.broadcasted_iota(jnp.int32, (CH, 1), 0) + c * CH
        before = (srow > sc) | ((srow == sc) & (irow < ic))
        cnt = jnp.sum(before.astype(jnp.float32), axis=1, keepdims=True)
        mask_chunks.append((cnt < KSEL).astype(jnp.float32))
    mask = jnp.concatenate(mask_chunks, axis=0)

    ltri = (lax.broadcasted_iota(jnp.int32, (CH, CH), 1) <
            lax.broadcasted_iota(jnp.int32, (CH, CH), 0)).astype(jnp.float32)
    off = jnp.zeros((1, 1), jnp.float32)
    slot_chunks = []
    for c in range(nch):
        mc = mask[c * CH:(c + 1) * CH]
        slot_chunks.append(
            jnp.dot(ltri, mc, preferred_element_type=jnp.float32) + off)
        off = off + jnp.sum(mc, keepdims=True)
    slot = jnp.concatenate(slot_chunks, axis=0)

    mrow = _row(mask)
    strow = _row(slot)
    mio = lax.broadcasted_iota(jnp.int32, (KSEL, SEGF), 0).astype(jnp.float32)
    P = jnp.where((strow == mio) & (mrow > 0.5), 1.0, 0.0).astype(jnp.bfloat16)
    xsel_ref[0] = jnp.dot(P, x2.astype(jnp.bfloat16),
                          preferred_element_type=jnp.float32).astype(jnp.bfloat16)
    mask_ref[0] = mask
    slot_ref[0] = slot


def _attn_kernel(xs_ref, wq_ref, wk_ref, wv_ref, wo_ref, g_ref,
                 w1_ref, b1_ref, w2_ref, b2_ref, y_ref):
    xs = xs_ref[0]
    ntok = xs.shape[0]
    q = jnp.dot(xs, wq_ref[...], preferred_element_type=jnp.float32)
    k = jnp.dot(xs, wk_ref[...], preferred_element_type=jnp.float32)
    v = jnp.dot(xs, wv_ref[...], preferred_element_type=jnp.float32)
    g = jax.nn.sigmoid(g_ref[...])
    yacc = jnp.zeros((ntok, D), jnp.float32)
    for hh in range(NH):
        qh = q[:, hh * DK:(hh + 1) * DK]
        kh = k[:, hh * DK:(hh + 1) * DK]
        vh = v[:, hh * DV:(hh + 1) * DV]
        gh = g[hh:hh + 1, :]
        mem = jnp.zeros((DK, DV), jnp.float32)
        zrow = jnp.full((1, DK), 1.0 / DK, jnp.float32)
        outs = []
        for s0 in range(0, ntok, SEGA):
            qs = qh[s0:s0 + SEGA]
            ks = kh[s0:s0 + SEGA]
            vs = vh[s0:s0 + SEGA]
            vsb = vs.astype(jnp.bfloat16)
            sq = jnp.where(qs > 0, qs + 1.0, jnp.exp(qs))
            num = jnp.dot(sq.astype(jnp.bfloat16),
                          mem.astype(jnp.bfloat16),
                          preferred_element_type=jnp.float32)
            den = lax.dot_general(sq, zrow, (((1,), (1,)), ((), ())),
                                  preferred_element_type=jnp.float32)
            att_mem = num / den
            sc_ = lax.dot_general(qs.astype(jnp.bfloat16),
                                  ks.astype(jnp.bfloat16),
                                  (((1,), (1,)), ((), ())),
                                  preferred_element_type=jnp.float32)
            sc_ = sc_ * (DK ** -0.5)
            mx = jnp.max(sc_, axis=1, keepdims=True)
            e = jnp.exp(sc_ - mx)
            att = e / jnp.sum(e, axis=1, keepdims=True)
            att_dot = jnp.dot(att.astype(jnp.bfloat16), vsb,
                              preferred_element_type=jnp.float32)
            sk = jnp.where(ks > 0, ks + 1.0, jnp.exp(ks))
            outs.append(gh * att_mem + (1.0 - gh) * att_dot)
            mem = mem + lax.dot_general(sk.astype(jnp.bfloat16), vsb,
                                        (((0,), (0,)), ((), ())),
                                        preferred_element_type=jnp.float32)
            zrow = zrow + jnp.sum(sk, axis=0, keepdims=True)
        oh = jnp.concatenate(outs, axis=0)
        yacc = yacc + jnp.dot(oh.astype(jnp.bfloat16),
                              wo_ref[hh * DV:(hh + 1) * DV, :],
                              preferred_element_type=jnp.float32)
    for c0 in range(0, ntok, 256):
        t = yacc[c0:c0 + 256].astype(jnp.bfloat16)
        hdn = jnp.maximum(
            jnp.dot(t, w1_ref[...], preferred_element_type=jnp.float32)
            + b1_ref[...], 0.0)
        y_ref[0, c0:c0 + 256] = (
            jnp.dot(hdn.astype(jnp.bfloat16), w2_ref[...],
                    preferred_element_type=jnp.float32) + b2_ref[...])


def _scatter_ln_kernel(x_ref, y_ref, mask_ref, slot_ref,
                       gamma_ref, beta_ref, o_ref):
    x2 = x_ref[0]
    yseg = y_ref[0]
    mask = mask_ref[0]
    slot = slot_ref[0]
    mio = lax.broadcasted_iota(jnp.int32, (SEGF, KSEL), 1).astype(jnp.float32)
    pt = jnp.where((slot == mio) & (mask > 0.5), 1.0, 0.0).astype(jnp.bfloat16)
    xu = x2 + jnp.dot(pt, yseg.astype(jnp.bfloat16),
                      preferred_element_type=jnp.float32)
    mu = jnp.mean(xu, axis=1, keepdims=True)
    xc = xu - mu
    var = jnp.mean(xc * xc, axis=1, keepdims=True)
    o_ref[0] = xc * lax.rsqrt(var + 1e-5) * gamma_ref[...] + beta_ref[...]


def kernel(x, Wq, Wk, Wv, Wo, betas, W1, b1, W2, b2, gamma, beta_ln,
           Ws1, bs1, Ws2, bs2):
    B_, S_, D_ = x.shape
    nseg = S_ // SEGF
    G = B_ * nseg
    xg = x.reshape(G, SEGF, D_)

    xsel, mask, slot = pl.pallas_call(
        _route_kernel,
        grid=(G,),
        in_specs=[
            pl.BlockSpec((1, SEGF, D_), lambda i: (i, 0, 0)),
            pl.BlockSpec((D_, SHID), lambda i: (0, 0)),
            pl.BlockSpec((1, SHID), lambda i: (0, 0)),
            pl.BlockSpec((SHID, 1), lambda i: (0, 0)),
            pl.BlockSpec((1, 1), lambda i: (0, 0)),
        ],
        out_specs=[
            pl.BlockSpec((1, KSEL, D_), lambda i: (i, 0, 0)),
            pl.BlockSpec((1, SEGF, 1), lambda i: (i, 0, 0)),
            pl.BlockSpec((1, SEGF, 1), lambda i: (i, 0, 0)),
        ],
        out_shape=[
            jax.ShapeDtypeStruct((G, KSEL, D_), jnp.bfloat16),
            jax.ShapeDtypeStruct((G, SEGF, 1), jnp.float32),
            jax.ShapeDtypeStruct((G, SEGF, 1), jnp.float32),
        ],
    )(xg, Ws1, bs1.reshape(1, SHID), Ws2, bs2.reshape(1, 1))

    xsel_b = xsel.reshape(B_, nseg * KSEL, D_)
    y = pl.pallas_call(
        _attn_kernel,
        grid=(B_,),
        in_specs=[
            pl.BlockSpec((1, nseg * KSEL, D_), lambda i: (i, 0, 0)),
            pl.BlockSpec((D_, NH * DK), lambda i: (0, 0)),
            pl.BlockSpec((D_, NH * DK), lambda i: (0, 0)),
            pl.BlockSpec((D_, NH * DV), lambda i: (0, 0)),
            pl.BlockSpec((NH * DV, D_), lambda i: (0, 0)),
            pl.BlockSpec((NH, DV), lambda i: (0, 0)),
            pl.BlockSpec((D_, DH), lambda i: (0, 0)),
            pl.BlockSpec((1, DH), lambda i: (0, 0)),
            pl.BlockSpec((DH, D_), lambda i: (0, 0)),
            pl.BlockSpec((1, D_), lambda i: (0, 0)),
        ],
        out_specs=pl.BlockSpec((1, nseg * KSEL, D_), lambda i: (i, 0, 0)),
        out_shape=jax.ShapeDtypeStruct((B_, nseg * KSEL, D_), jnp.float32),
    )(xsel_b, Wq.astype(jnp.bfloat16), Wk.astype(jnp.bfloat16),
      Wv.astype(jnp.bfloat16), Wo.astype(jnp.bfloat16), betas.reshape(NH, DV),
      W1.astype(jnp.bfloat16), b1.reshape(1, DH),
      W2.astype(jnp.bfloat16), b2.reshape(1, D_))

    yf = y.reshape(G, KSEL, D_)

    out = pl.pallas_call(
        _scatter_ln_kernel,
        grid=(G,),
        in_specs=[
            pl.BlockSpec((1, SEGF, D_), lambda i: (i, 0, 0)),
            pl.BlockSpec((1, KSEL, D_), lambda i: (i, 0, 0)),
            pl.BlockSpec((1, SEGF, 1), lambda i: (i, 0, 0)),
            pl.BlockSpec((1, SEGF, 1), lambda i: (i, 0, 0)),
            pl.BlockSpec((1, D_), lambda i: (0, 0)),
            pl.BlockSpec((1, D_), lambda i: (0, 0)),
        ],
        out_specs=pl.BlockSpec((1, SEGF, D_), lambda i: (i, 0, 0)),
        out_shape=jax.ShapeDtypeStruct((G, SEGF, D_), jnp.float32),
    )(xg, yf, mask, slot, gamma.reshape(1, D_), beta_ln.reshape(1, D_))

    return out.reshape(B_, S_, D_), mask.reshape(B_ * S_, 1)


# speedup vs baseline: 1.0823x; 1.0823x over previous
import jax
import jax.numpy as jnp
from jax import lax
from jax.experimental import pallas as pl

D = 768
DH = 3072
DK = 64
DV = 64
NH = 12
SEGF = 2048
KSEL = 256
SEGA = 256
SHID = 256
CH = 256


def _route_kernel(x_ref, ws1_ref, bs1_ref, ws2_ref, bs2_ref,
                  xsel_ref, mask_ref, slot_ref):
    x2 = x_ref[0]
    h = jnp.maximum(
        jnp.dot(x2, ws1_ref[...], preferred_element_type=jnp.float32)
        + bs1_ref[...], 0.0)
    scol = (jnp.dot(h, ws2_ref[...], preferred_element_type=jnp.float32)
            + bs2_ref[...])

    nch = SEGF // CH
    eye = (lax.broadcasted_iota(jnp.int32, (CH, CH), 0) ==
           lax.broadcasted_iota(jnp.int32, (CH, CH), 1)).astype(jnp.float32)

    def _row(vcol):
        parts = [
            lax.dot_general(vcol[c * CH:(c + 1) * CH], eye,
                            (((0,), (0,)), ((), ())),
                            preferred_element_type=jnp.float32)
            for c in range(nch)
        ]
        return jnp.concatenate(parts, axis=1)

    eye_b = eye.astype(jnp.bfloat16)

    def _rowb(vcol_b):
        parts = [
            lax.dot_general(vcol_b[c * CH:(c + 1) * CH], eye_b,
                            (((0,), (0,)), ((), ())),
                            preferred_element_type=jnp.float32)
            for c in range(nch)
        ]
        return jnp.concatenate(parts, axis=1)

    sa = scol.astype(jnp.bfloat16)
    sbp = scol - sa.astype(jnp.float32)
    sb = sbp.astype(jnp.bfloat16)
    sc2 = (sbp - sb.astype(jnp.float32)).astype(jnp.bfloat16)
    srow = (_rowb(sb) + _rowb(sc2)) + _rowb(sa)

    irow = lax.broadcasted_iota(jnp.int32, (1, SEGF), 1)

    mask_chunks = []
    for c in range(nch):
        sc = scol[c * CH:(c + 1) * CH]
        ic = lax.broadcasted_iota(jnp.int32, (CH, 1), 0) + c * CH
        before = (srow > sc) | ((srow == sc) & (irow < ic))
        cnt = jnp.sum(before.astype(jnp.float32), axis=1, keepdims=True)
        mask_chunks.append((cnt < KSEL).astype(jnp.float32))
    mask = jnp.concatenate(mask_chunks, axis=0)

    ltri = (lax.broadcasted_iota(jnp.int32, (CH, CH), 1) <
            lax.broadcasted_iota(jnp.int32, (CH, CH), 0)).astype(jnp.float32)
    off = jnp.zeros((1, 1), jnp.float32)
    slot_chunks = []
    for c in range(nch):
        mc = mask[c * CH:(c + 1) * CH]
        slot_chunks.append(
            jnp.dot(ltri, mc, preferred_element_type=jnp.float32) + off)
        off = off + jnp.sum(mc, keepdims=True)
    slot = jnp.concatenate(slot_chunks, axis=0)

    mrow = _row(mask)
    strow = _row(slot)
    mio = lax.broadcasted_iota(jnp.int32, (KSEL, SEGF), 0).astype(jnp.float32)
    P = jnp.where((strow == mio) & (mrow > 0.5), 1.0, 0.0).astype(jnp.bfloat16)
    xsel_ref[0] = jnp.dot(P, x2.astype(jnp.bfloat16),
                          preferred_element_type=jnp.float32).astype(jnp.bfloat16)
    mask_ref[0] = mask
    slot_ref[0] = slot


def _attn_kernel(xs_ref, wq_ref, wk_ref, wv_ref, wo_ref, g_ref, y_ref):
    xs = xs_ref[0]
    ntok = xs.shape[0]
    q = jnp.dot(xs, wq_ref[...], preferred_element_type=jnp.float32)
    k = jnp.dot(xs, wk_ref[...], preferred_element_type=jnp.float32)
    v = jnp.dot(xs, wv_ref[...], preferred_element_type=jnp.float32)
    g = jax.nn.sigmoid(g_ref[...])
    yacc = jnp.zeros((ntok, D), jnp.float32)
    for hh in range(NH):
        qh = q[:, hh * DK:(hh + 1) * DK]
        kh = k[:, hh * DK:(hh + 1) * DK]
        vh = v[:, hh * DV:(hh + 1) * DV]
        gh = g[hh:hh + 1, :]
        mem = jnp.zeros((DK, DV), jnp.float32)
        zrow = jnp.full((1, DK), 1.0 / DK, jnp.float32)
        outs = []
        for s0 in range(0, ntok, SEGA):
            qs = qh[s0:s0 + SEGA]
            ks = kh[s0:s0 + SEGA]
            vs = vh[s0:s0 + SEGA]
            vsb = vs.astype(jnp.bfloat16)
            sq = jnp.where(qs > 0, qs + 1.0, jnp.exp(qs))
            num = jnp.dot(sq.astype(jnp.bfloat16),
                          mem.astype(jnp.bfloat16),
                          preferred_element_type=jnp.float32)
            den = lax.dot_general(sq, zrow, (((1,), (1,)), ((), ())),
                                  preferred_element_type=jnp.float32)
            att_mem = num / den
            sc_ = lax.dot_general(qs.astype(jnp.bfloat16),
                                  ks.astype(jnp.bfloat16),
                                  (((1,), (1,)), ((), ())),
                                  preferred_element_type=jnp.float32)
            sc_ = sc_ * (DK ** -0.5)
            mx = jnp.max(sc_, axis=1, keepdims=True)
            e = jnp.exp(sc_ - mx)
            att = e / jnp.sum(e, axis=1, keepdims=True)
            att_dot = jnp.dot(att.astype(jnp.bfloat16), vsb,
                              preferred_element_type=jnp.float32)
            sk = jnp.where(ks > 0, ks + 1.0, jnp.exp(ks))
            outs.append(gh * att_mem + (1.0 - gh) * att_dot)
            mem = mem + lax.dot_general(sk.astype(jnp.bfloat16), vsb,
                                        (((0,), (0,)), ((), ())),
                                        preferred_element_type=jnp.float32)
            zrow = zrow + jnp.sum(sk, axis=0, keepdims=True)
        oh = jnp.concatenate(outs, axis=0)
        yacc = yacc + jnp.dot(oh.astype(jnp.bfloat16),
                              wo_ref[hh * DV:(hh + 1) * DV, :],
                              preferred_element_type=jnp.float32)
    y_ref[0] = yacc


def _ffn_kernel(y_ref, w1_ref, b1_ref, w2_ref, b2_ref, o_ref):
    t = y_ref[0].astype(jnp.bfloat16)
    hdn = jnp.maximum(
        jnp.dot(t, w1_ref[...], preferred_element_type=jnp.float32)
        + b1_ref[...], 0.0)
    o_ref[0] = (jnp.dot(hdn.astype(jnp.bfloat16), w2_ref[...],
                        preferred_element_type=jnp.float32)
                + b2_ref[...])


def _scatter_ln_kernel(x_ref, y_ref, mask_ref, slot_ref,
                       gamma_ref, beta_ref, o_ref):
    x2 = x_ref[0]
    yseg = y_ref[0]
    mask = mask_ref[0]
    slot = slot_ref[0]
    mio = lax.broadcasted_iota(jnp.int32, (SEGF, KSEL), 1).astype(jnp.float32)
    pt = jnp.where((slot == mio) & (mask > 0.5), 1.0, 0.0).astype(jnp.bfloat16)
    xu = x2 + jnp.dot(pt, yseg.astype(jnp.bfloat16),
                      preferred_element_type=jnp.float32)
    mu = jnp.mean(xu, axis=1, keepdims=True)
    xc = xu - mu
    var = jnp.mean(xc * xc, axis=1, keepdims=True)
    o_ref[0] = xc * lax.rsqrt(var + 1e-5) * gamma_ref[...] + beta_ref[...]


def kernel(x, Wq, Wk, Wv, Wo, betas, W1, b1, W2, b2, gamma, beta_ln,
           Ws1, bs1, Ws2, bs2):
    B_, S_, D_ = x.shape
    nseg = S_ // SEGF
    G = B_ * nseg
    xg = x.reshape(G, SEGF, D_)

    xsel, mask, slot = pl.pallas_call(
        _route_kernel,
        grid=(G,),
        in_specs=[
            pl.BlockSpec((1, SEGF, D_), lambda i: (i, 0, 0)),
            pl.BlockSpec((D_, SHID), lambda i: (0, 0)),
            pl.BlockSpec((1, SHID), lambda i: (0, 0)),
            pl.BlockSpec((SHID, 1), lambda i: (0, 0)),
            pl.BlockSpec((1, 1), lambda i: (0, 0)),
        ],
        out_specs=[
            pl.BlockSpec((1, KSEL, D_), lambda i: (i, 0, 0)),
            pl.BlockSpec((1, SEGF, 1), lambda i: (i, 0, 0)),
            pl.BlockSpec((1, SEGF, 1), lambda i: (i, 0, 0)),
        ],
        out_shape=[
            jax.ShapeDtypeStruct((G, KSEL, D_), jnp.bfloat16),
            jax.ShapeDtypeStruct((G, SEGF, 1), jnp.float32),
            jax.ShapeDtypeStruct((G, SEGF, 1), jnp.float32),
        ],
    )(xg, Ws1, bs1.reshape(1, SHID), Ws2, bs2.reshape(1, 1))

    xsel_b = xsel.reshape(B_, nseg * KSEL, D_)
    y = pl.pallas_call(
        _attn_kernel,
        grid=(B_,),
        in_specs=[
            pl.BlockSpec((1, nseg * KSEL, D_), lambda i: (i, 0, 0)),
            pl.BlockSpec((D_, NH * DK), lambda i: (0, 0)),
            pl.BlockSpec((D_, NH * DK), lambda i: (0, 0)),
            pl.BlockSpec((D_, NH * DV), lambda i: (0, 0)),
            pl.BlockSpec((NH * DV, D_), lambda i: (0, 0)),
            pl.BlockSpec((NH, DV), lambda i: (0, 0)),
        ],
        out_specs=pl.BlockSpec((1, nseg * KSEL, D_), lambda i: (i, 0, 0)),
        out_shape=jax.ShapeDtypeStruct((B_, nseg * KSEL, D_), jnp.float32),
    )(xsel_b, Wq.astype(jnp.bfloat16), Wk.astype(jnp.bfloat16),
      Wv.astype(jnp.bfloat16), Wo.astype(jnp.bfloat16), betas.reshape(NH, DV))

    yg = y.reshape(G, KSEL, D_)
    yf = pl.pallas_call(
        _ffn_kernel,
        grid=(G,),
        in_specs=[
            pl.BlockSpec((1, KSEL, D_), lambda i: (i, 0, 0)),
            pl.BlockSpec((D_, DH), lambda i: (0, 0)),
            pl.BlockSpec((1, DH), lambda i: (0, 0)),
            pl.BlockSpec((DH, D_), lambda i: (0, 0)),
            pl.BlockSpec((1, D_), lambda i: (0, 0)),
        ],
        out_specs=pl.BlockSpec((1, KSEL, D_), lambda i: (i, 0, 0)),
        out_shape=jax.ShapeDtypeStruct((G, KSEL, D_), jnp.float32),
    )(yg, W1.astype(jnp.bfloat16), b1.reshape(1, DH),
      W2.astype(jnp.bfloat16), b2.reshape(1, D_))

    out = pl.pallas_call(
        _scatter_ln_kernel,
        grid=(G,),
        in_specs=[
            pl.BlockSpec((1, SEGF, D_), lambda i: (i, 0, 0)),
            pl.BlockSpec((1, KSEL, D_), lambda i: (i, 0, 0)),
            pl.BlockSpec((1, SEGF, 1), lambda i: (i, 0, 0)),
            pl.BlockSpec((1, SEGF, 1), lambda i: (i, 0, 0)),
            pl.BlockSpec((1, D_), lambda i: (0, 0)),
            pl.BlockSpec((1, D_), lambda i: (0, 0)),
        ],
        out_specs=pl.BlockSpec((1, SEGF, D_), lambda i: (i, 0, 0)),
        out_shape=jax.ShapeDtypeStruct((G, SEGF, D_), jnp.float32),
    )(xg, yf, mask, slot, gamma.reshape(1, D_), beta_ln.reshape(1, D_))

    return out.reshape(B_, S_, D_), mask.reshape(B_ * S_, 1)


# speedup vs baseline: 1.1401x; 1.0534x over previous
import jax
import jax.numpy as jnp
from jax import lax
from jax.experimental import pallas as pl

D = 768
DH = 3072
DK = 64
DV = 64
NH = 12
SEGF = 2048
KSEL = 256
SEGA = 256
SHID = 256
CH = 256


def _route_kernel(x_ref, ws1_ref, bs1_ref, ws2_ref, bs2_ref,
                  xsel_ref, mask_ref, slot_ref):
    x2 = x_ref[0]
    h = jnp.maximum(
        jnp.dot(x2, ws1_ref[...], preferred_element_type=jnp.float32)
        + bs1_ref[...], 0.0)
    scol = (jnp.dot(h, ws2_ref[...], preferred_element_type=jnp.float32)
            + bs2_ref[...])

    nch = SEGF // CH
    eye = (lax.broadcasted_iota(jnp.int32, (CH, CH), 0) ==
           lax.broadcasted_iota(jnp.int32, (CH, CH), 1)).astype(jnp.float32)

    def _row(vcol):
        parts = [
            lax.dot_general(vcol[c * CH:(c + 1) * CH], eye,
                            (((0,), (0,)), ((), ())),
                            preferred_element_type=jnp.float32)
            for c in range(nch)
        ]
        return jnp.concatenate(parts, axis=1)

    eye_b = eye.astype(jnp.bfloat16)

    def _rowb(vcol_b):
        parts = [
            lax.dot_general(vcol_b[c * CH:(c + 1) * CH], eye_b,
                            (((0,), (0,)), ((), ())),
                            preferred_element_type=jnp.float32)
            for c in range(nch)
        ]
        return jnp.concatenate(parts, axis=1)

    sa = scol.astype(jnp.bfloat16)
    sbp = scol - sa.astype(jnp.float32)
    sb = sbp.astype(jnp.bfloat16)
    sc2 = (sbp - sb.astype(jnp.float32)).astype(jnp.bfloat16)
    srow = (_rowb(sb) + _rowb(sc2)) + _rowb(sa)

    irow = lax.broadcasted_iota(jnp.int32, (1, SEGF), 1)

    mask_chunks = []
    for c in range(nch):
        sc = scol[c * CH:(c + 1) * CH]
        ic = lax.broadcasted_iota(jnp.int32, (CH, 1), 0) + c * CH
        before = (srow > sc) | ((srow == sc) & (irow < ic))
        cnt = jnp.sum(before.astype(jnp.float32), axis=1, keepdims=True)
        mask_chunks.append((cnt < KSEL).astype(jnp.float32))
    mask = jnp.concatenate(mask_chunks, axis=0)

    ltri = (lax.broadcasted_iota(jnp.int32, (CH, CH), 1) <
            lax.broadcasted_iota(jnp.int32, (CH, CH), 0)).astype(jnp.float32)
    off = jnp.zeros((1, 1), jnp.float32)
    slot_chunks = []
    for c in range(nch):
        mc = mask[c * CH:(c + 1) * CH]
        slot_chunks.append(
            jnp.dot(ltri, mc, preferred_element_type=jnp.float32) + off)
        off = off + jnp.sum(mc, keepdims=True)
    slot = jnp.concatenate(slot_chunks, axis=0)

    mrow = _row(mask)
    strow = _row(slot)
    mio = lax.broadcasted_iota(jnp.int32, (KSEL, SEGF), 0).astype(jnp.float32)
    P = jnp.where((strow == mio) & (mrow > 0.5), 1.0, 0.0).astype(jnp.bfloat16)
    xsel_ref[0] = jnp.dot(P, x2.astype(jnp.bfloat16),
                          preferred_element_type=jnp.float32).astype(jnp.bfloat16)
    mask_ref[0] = mask
    slot_ref[0] = slot


def _attn_kernel(xs_ref, wq_ref, wk_ref, wv_ref, wo_ref, g_ref, y_ref):
    xs = xs_ref[0]
    ntok = xs.shape[0]
    q = jnp.dot(xs, wq_ref[...].astype(jnp.bfloat16),
                preferred_element_type=jnp.float32)
    k = jnp.dot(xs, wk_ref[...].astype(jnp.bfloat16),
                preferred_element_type=jnp.float32)
    v = jnp.dot(xs, wv_ref[...].astype(jnp.bfloat16),
                preferred_element_type=jnp.float32)
    g = jax.nn.sigmoid(g_ref[...])
    yacc = jnp.zeros((ntok, D), jnp.float32)
    for hh in range(NH):
        qh = q[:, hh * DK:(hh + 1) * DK]
        kh = k[:, hh * DK:(hh + 1) * DK]
        vh = v[:, hh * DV:(hh + 1) * DV]
        gh = g[hh:hh + 1, :]
        mem = jnp.zeros((DK, DV), jnp.float32)
        zrow = jnp.full((1, DK), 1.0 / DK, jnp.float32)
        outs = []
        for s0 in range(0, ntok, SEGA):
            qs = qh[s0:s0 + SEGA]
            ks = kh[s0:s0 + SEGA]
            vs = vh[s0:s0 + SEGA]
            vsb = vs.astype(jnp.bfloat16)
            sq = jnp.where(qs > 0, qs + 1.0, jnp.exp(qs))
            num = jnp.dot(sq.astype(jnp.bfloat16),
                          mem.astype(jnp.bfloat16),
                          preferred_element_type=jnp.float32)
            den = lax.dot_general(sq, zrow, (((1,), (1,)), ((), ())),
                                  preferred_element_type=jnp.float32)
            att_mem = num / den
            sc_ = lax.dot_general(qs.astype(jnp.bfloat16),
                                  ks.astype(jnp.bfloat16),
                                  (((1,), (1,)), ((), ())),
                                  preferred_element_type=jnp.float32)
            sc_ = sc_ * (DK ** -0.5)
            mx = jnp.max(sc_, axis=1, keepdims=True)
            e = jnp.exp(sc_ - mx)
            att = e / jnp.sum(e, axis=1, keepdims=True)
            att_dot = jnp.dot(att.astype(jnp.bfloat16), vsb,
                              preferred_element_type=jnp.float32)
            sk = jnp.where(ks > 0, ks + 1.0, jnp.exp(ks))
            outs.append(gh * att_mem + (1.0 - gh) * att_dot)
            mem = mem + lax.dot_general(sk.astype(jnp.bfloat16), vsb,
                                        (((0,), (0,)), ((), ())),
                                        preferred_element_type=jnp.float32)
            zrow = zrow + jnp.sum(sk, axis=0, keepdims=True)
        oh = jnp.concatenate(outs, axis=0)
        yacc = yacc + jnp.dot(oh.astype(jnp.bfloat16),
                              wo_ref[hh * DV:(hh + 1) * DV, :].astype(jnp.bfloat16),
                              preferred_element_type=jnp.float32)
    y_ref[0] = yacc


def _ffn_kernel(y_ref, w1_ref, b1_ref, w2_ref, b2_ref, o_ref):
    t = y_ref[0].astype(jnp.bfloat16)
    hdn = jnp.maximum(
        jnp.dot(t, w1_ref[...].astype(jnp.bfloat16),
                preferred_element_type=jnp.float32)
        + b1_ref[...], 0.0)
    o_ref[0] = (jnp.dot(hdn.astype(jnp.bfloat16),
                        w2_ref[...].astype(jnp.bfloat16),
                        preferred_element_type=jnp.float32)
                + b2_ref[...])


def _scatter_ln_kernel(x_ref, y_ref, mask_ref, slot_ref,
                       gamma_ref, beta_ref, o_ref):
    x2 = x_ref[0]
    yseg = y_ref[0]
    mask = mask_ref[0]
    slot = slot_ref[0]
    mio = lax.broadcasted_iota(jnp.int32, (SEGF, KSEL), 1).astype(jnp.float32)
    pt = jnp.where((slot == mio) & (mask > 0.5), 1.0, 0.0).astype(jnp.bfloat16)
    xu = x2 + jnp.dot(pt, yseg.astype(jnp.bfloat16),
                      preferred_element_type=jnp.float32)
    mu = jnp.mean(xu, axis=1, keepdims=True)
    xc = xu - mu
    var = jnp.mean(xc * xc, axis=1, keepdims=True)
    o_ref[0] = xc * lax.rsqrt(var + 1e-5) * gamma_ref[...] + beta_ref[...]


def kernel(x, Wq, Wk, Wv, Wo, betas, W1, b1, W2, b2, gamma, beta_ln,
           Ws1, bs1, Ws2, bs2):
    B_, S_, D_ = x.shape
    nseg = S_ // SEGF
    G = B_ * nseg
    xg = x.reshape(G, SEGF, D_)

    xsel, mask, slot = pl.pallas_call(
        _route_kernel,
        grid=(G,),
        in_specs=[
            pl.BlockSpec((1, SEGF, D_), lambda i: (i, 0, 0)),
            pl.BlockSpec((D_, SHID), lambda i: (0, 0)),
            pl.BlockSpec((1, SHID), lambda i: (0, 0)),
            pl.BlockSpec((SHID, 1), lambda i: (0, 0)),
            pl.BlockSpec((1, 1), lambda i: (0, 0)),
        ],
        out_specs=[
            pl.BlockSpec((1, KSEL, D_), lambda i: (i, 0, 0)),
            pl.BlockSpec((1, SEGF, 1), lambda i: (i, 0, 0)),
            pl.BlockSpec((1, SEGF, 1), lambda i: (i, 0, 0)),
        ],
        out_shape=[
            jax.ShapeDtypeStruct((G, KSEL, D_), jnp.bfloat16),
            jax.ShapeDtypeStruct((G, SEGF, 1), jnp.float32),
            jax.ShapeDtypeStruct((G, SEGF, 1), jnp.float32),
        ],
    )(xg, Ws1, bs1.reshape(1, SHID), Ws2, bs2.reshape(1, 1))

    xsel_b = xsel.reshape(B_, nseg * KSEL, D_)
    y = pl.pallas_call(
        _attn_kernel,
        grid=(B_,),
        in_specs=[
            pl.BlockSpec((1, nseg * KSEL, D_), lambda i: (i, 0, 0)),
            pl.BlockSpec((D_, NH * DK), lambda i: (0, 0)),
            pl.BlockSpec((D_, NH * DK), lambda i: (0, 0)),
            pl.BlockSpec((D_, NH * DV), lambda i: (0, 0)),
            pl.BlockSpec((NH * DV, D_), lambda i: (0, 0)),
            pl.BlockSpec((NH, DV), lambda i: (0, 0)),
        ],
        out_specs=pl.BlockSpec((1, nseg * KSEL, D_), lambda i: (i, 0, 0)),
        out_shape=jax.ShapeDtypeStruct((B_, nseg * KSEL, D_), jnp.float32),
    )(xsel_b, Wq, Wk, Wv, Wo, betas.reshape(NH, DV))

    yg = y.reshape(G, KSEL, D_)
    yf = pl.pallas_call(
        _ffn_kernel,
        grid=(G,),
        in_specs=[
            pl.BlockSpec((1, KSEL, D_), lambda i: (i, 0, 0)),
            pl.BlockSpec((D_, DH), lambda i: (0, 0)),
            pl.BlockSpec((1, DH), lambda i: (0, 0)),
            pl.BlockSpec((DH, D_), lambda i: (0, 0)),
            pl.BlockSpec((1, D_), lambda i: (0, 0)),
        ],
        out_specs=pl.BlockSpec((1, KSEL, D_), lambda i: (i, 0, 0)),
        out_shape=jax.ShapeDtypeStruct((G, KSEL, D_), jnp.float32),
    )(yg, W1, b1.reshape(1, DH), W2, b2.reshape(1, D_))

    out = pl.pallas_call(
        _scatter_ln_kernel,
        grid=(G,),
        in_specs=[
            pl.BlockSpec((1, SEGF, D_), lambda i: (i, 0, 0)),
            pl.BlockSpec((1, KSEL, D_), lambda i: (i, 0, 0)),
            pl.BlockSpec((1, SEGF, 1), lambda i: (i, 0, 0)),
            pl.BlockSpec((1, SEGF, 1), lambda i: (i, 0, 0)),
            pl.BlockSpec((1, D_), lambda i: (0, 0)),
            pl.BlockSpec((1, D_), lambda i: (0, 0)),
        ],
        out_specs=pl.BlockSpec((1, SEGF, D_), lambda i: (i, 0, 0)),
        out_shape=jax.ShapeDtypeStruct((G, SEGF, D_), jnp.float32),
    )(xg, yf, mask, slot, gamma.reshape(1, D_), beta_ln.reshape(1, D_))

    return out.reshape(B_, S_, D_), mask.reshape(B_ * S_, 1)
